# baseline (speedup 1.0000x reference)
import functools

import jax
import jax.numpy as jnp
from jax import lax
from jax.experimental import pallas as pl
from jax.experimental.pallas import tpu as pltpu
from jax.experimental.pallas import tpu_sc as plsc

N_OBJ = 128
D = 64
TB = 100
BB = 2

_ROWS_PER_WORKER = 128
_N_WORKERS = N_OBJ // _ROWS_PER_WORKER


def _sc_gather(table_hbm, idx_hbm, out_hbm, idx_v, rows_v, sem):
    info = plsc.get_sparse_core_info()
    wid = lax.axis_index("s") * info.num_cores + lax.axis_index("c")

    @pl.when(wid < _N_WORKERS)
    def _():
        base = wid * _ROWS_PER_WORKER
        pltpu.sync_copy(idx_hbm.at[pl.ds(base, _ROWS_PER_WORKER)], idx_v)
        pltpu.async_copy(table_hbm.at[idx_v], rows_v, sem).wait()
        pltpu.sync_copy(rows_v, out_hbm.at[pl.ds(base, _ROWS_PER_WORKER)])


def _gather_emb(table, obj_ids):
    mesh = plsc.VectorSubcoreMesh(core_axis_name="c", subcore_axis_name="s")
    kern = functools.partial(
        pl.kernel,
        mesh=mesh,
        out_type=jax.ShapeDtypeStruct((N_OBJ, 2 * D), jnp.float32),
        scratch_types=[
            pltpu.VMEM((_ROWS_PER_WORKER,), jnp.int32),
            pltpu.VMEM((_ROWS_PER_WORKER, 2 * D), jnp.float32),
            pltpu.SemaphoreType.DMA,
        ],
    )(_sc_gather)
    table128 = jnp.pad(table, ((0, 0), (0, 2 * D - table.shape[1])))
    return kern(table128, obj_ids)


def _concat_body(emb_ref, s_ref, o_ref):
    emb = emb_ref[:, 0:D]
    st = jnp.swapaxes(s_ref[...], 2, 3)
    embb = jnp.broadcast_to(emb[None, None], (BB, TB, N_OBJ, D))
    o_ref[:, :, :, 0:D] = st
    o_ref[:, :, :, D:2 * D] = embb


def kernel(states, table, obj_ids):
    B, N, t, d = states.shape
    emb = _gather_emb(table, obj_ids)
    s_t = states.transpose(0, 2, 3, 1)
    out_t = pl.pallas_call(
        _concat_body,
        grid=(B // BB, t // TB),
        in_specs=[
            pl.BlockSpec((N, 2 * d), lambda g, h: (0, 0)),
            pl.BlockSpec((BB, TB, d, N), lambda g, h: (g, h, 0, 0)),
        ],
        out_specs=pl.BlockSpec((BB, TB, N, 2 * d), lambda g, h: (g, h, 0, 0)),
        out_shape=jax.ShapeDtypeStruct((B, t, N, 2 * d), jnp.float32),
    )(emb, s_t)
    return out_t.transpose(0, 2, 1, 3)


# speedup vs baseline: 1.0077x; 1.0077x over previous
import functools

import jax
import jax.numpy as jnp
from jax import lax
from jax.experimental import pallas as pl
from jax.experimental.pallas import tpu as pltpu
from jax.experimental.pallas import tpu_sc as plsc

N_OBJ = 128
D = 64
TB = 50
BB = 4

_ROWS_PER_WORKER = 8
_N_WORKERS = N_OBJ // _ROWS_PER_WORKER


def _sc_gather(table_hbm, idx_hbm, out_hbm, idx_v, rows_v, sem):
    info = plsc.get_sparse_core_info()
    wid = lax.axis_index("s") * info.num_cores + lax.axis_index("c")

    @pl.when(wid < _N_WORKERS)
    def _():
        base = wid * _ROWS_PER_WORKER
        pltpu.sync_copy(idx_hbm.at[pl.ds(base, _ROWS_PER_WORKER)], idx_v)
        pltpu.async_copy(table_hbm.at[idx_v], rows_v, sem).wait()
        pltpu.sync_copy(rows_v, out_hbm.at[pl.ds(base, _ROWS_PER_WORKER)])


def _gather_emb(table, obj_ids):
    mesh = plsc.VectorSubcoreMesh(core_axis_name="c", subcore_axis_name="s")
    kern = functools.partial(
        pl.kernel,
        mesh=mesh,
        out_type=jax.ShapeDtypeStruct((N_OBJ, 2 * D), jnp.float32),
        scratch_types=[
            pltpu.VMEM((_ROWS_PER_WORKER,), jnp.int32),
            pltpu.VMEM((_ROWS_PER_WORKER, 2 * D), jnp.float32),
            pltpu.SemaphoreType.DMA,
        ],
    )(_sc_gather)
    table128 = jnp.pad(table, ((0, 0), (0, 2 * D - table.shape[1])))
    return kern(table128, obj_ids)


def _concat_body(emb_ref, s_ref, o_ref):
    emb = emb_ref[:, 0:D]
    st = jnp.swapaxes(s_ref[...], 2, 3)
    embb = jnp.broadcast_to(emb[None, None], (BB, TB, N_OBJ, D))
    o_ref[:, :, :, 0:D] = st
    o_ref[:, :, :, D:2 * D] = embb


def kernel(states, table, obj_ids):
    B, N, t, d = states.shape
    emb = _gather_emb(table, obj_ids)
    s_t = states.transpose(0, 2, 3, 1)
    out_t = pl.pallas_call(
        _concat_body,
        grid=(B // BB, t // TB),
        in_specs=[
            pl.BlockSpec((N, 2 * d), lambda g, h: (0, 0)),
            pl.BlockSpec((BB, TB, d, N), lambda g, h: (g, h, 0, 0)),
        ],
        out_specs=pl.BlockSpec((BB, TB, N, 2 * d), lambda g, h: (g, h, 0, 0)),
        out_shape=jax.ShapeDtypeStruct((B, t, N, 2 * d), jnp.float32),
    )(emb, s_t)
    return out_t.transpose(0, 2, 1, 3)


# speedup vs baseline: 1.0131x; 1.0053x over previous
import functools

import jax
import jax.numpy as jnp
from jax import lax
from jax.experimental import pallas as pl
from jax.experimental.pallas import tpu as pltpu
from jax.experimental.pallas import tpu_sc as plsc

N_OBJ = 128
D = 64
TB = 100
BB = 2

_ROWS_PER_WORKER = 8
_N_WORKERS = N_OBJ // _ROWS_PER_WORKER


def _sc_gather(table_hbm, idx_hbm, out_hbm, idx_v, rows_v, sem):
    info = plsc.get_sparse_core_info()
    wid = lax.axis_index("s") * info.num_cores + lax.axis_index("c")

    @pl.when(wid < _N_WORKERS)
    def _():
        base = wid * _ROWS_PER_WORKER
        pltpu.sync_copy(idx_hbm.at[pl.ds(base, _ROWS_PER_WORKER)], idx_v)
        pltpu.async_copy(table_hbm.at[idx_v], rows_v, sem).wait()
        pltpu.sync_copy(rows_v, out_hbm.at[pl.ds(base, _ROWS_PER_WORKER)])


def _gather_emb(table, obj_ids):
    mesh = plsc.VectorSubcoreMesh(core_axis_name="c", subcore_axis_name="s")
    kern = functools.partial(
        pl.kernel,
        mesh=mesh,
        out_type=jax.ShapeDtypeStruct((N_OBJ, 2 * D), jnp.float32),
        scratch_types=[
            pltpu.VMEM((_ROWS_PER_WORKER,), jnp.int32),
            pltpu.VMEM((_ROWS_PER_WORKER, 2 * D), jnp.float32),
            pltpu.SemaphoreType.DMA,
        ],
    )(_sc_gather)
    table128 = jnp.pad(table, ((0, 0), (0, 2 * D - table.shape[1])))
    return kern(table128, obj_ids)


def _concat_body(emb_ref, s_ref, o_ref):
    emb = emb_ref[:, 0:D]
    st = jnp.swapaxes(s_ref[...], 2, 3)
    embb = jnp.broadcast_to(emb[None, None], (BB, TB, N_OBJ, D))
    o_ref[:, :, :, 0:D] = st
    o_ref[:, :, :, D:2 * D] = embb


def kernel(states, table, obj_ids):
    B, N, t, d = states.shape
    emb = _gather_emb(table, obj_ids)
    s_t = states.transpose(0, 2, 3, 1)
    out_t = pl.pallas_call(
        _concat_body,
        grid=(B // BB, t // TB),
        in_specs=[
            pl.BlockSpec((N, 2 * d), lambda g, h: (0, 0)),
            pl.BlockSpec((BB, TB, d, N), lambda g, h: (g, h, 0, 0)),
        ],
        out_specs=pl.BlockSpec((BB, TB, N, 2 * d), lambda g, h: (g, h, 0, 0)),
        out_shape=jax.ShapeDtypeStruct((B, t, N, 2 * d), jnp.float32),
    )(emb, s_t)
    return out_t.transpose(0, 2, 1, 3)
